# baseline (speedup 1.0000x reference)
import functools
import jax
import jax.numpy as jnp
from jax import lax
from jax.experimental import pallas as pl
from jax.experimental.pallas import tpu as pltpu
from jax.experimental.pallas import tpu_sc as plsc

_K = 64
_N = 32768
_ROWS = 128
_BLOCK_ROWS = 32


def _search_body(x_ref, t_ref, j_ref):
    x = x_ref[...]
    r = x.shape[0]
    xi = jax.lax.bitcast_convert_type(x, jnp.int32)
    z = xi ^ ((xi >> 31) & jnp.int32(0x7FFFFFFF))

    def count_ge(t):
        ind = jnp.where(z >= t, jnp.float32(1.0), jnp.float32(0.0))
        return jnp.sum(ind, axis=1, keepdims=True)

    kf = jnp.float32(_K)
    one = jnp.full((r, 1), 1, jnp.int32)
    c1 = count_ge(one)

    zc = jnp.max(z.reshape(r, _N // 128, 128), axis=1)
    zmax = jnp.max(zc, axis=1, keepdims=True)

    def countc_ge(t):
        ind = jnp.where(zc >= t, jnp.float32(1.0), jnp.float32(0.0))
        return jnp.sum(ind, axis=1, keepdims=True)

    hi0 = jnp.maximum(zmax + 1, one + 1)

    def s1body(i, st):
        lo, hi = st
        q = (hi - lo) // 4
        m1 = lo + q
        m2 = jnp.maximum(lo + 2 * q, lo + 1)
        m3 = jnp.maximum(lo + 3 * q, m2)
        cc1 = countc_ge(m1)
        cc2 = countc_ge(m2)
        cc3 = countc_ge(m3)
        lo = jnp.where(cc1 >= kf, m1, lo)
        lo = jnp.where(cc2 >= kf, m2, lo)
        lo = jnp.where(cc3 >= kf, m3, lo)
        hi = jnp.where(cc3 < kf, m3, hi)
        hi = jnp.where(cc2 < kf, m2, hi)
        hi = jnp.where(cc1 < kf, m1, hi)
        return lo, hi

    s1lo, _ = jax.lax.fori_loop(0, 9, s1body, (one, hi0))

    done0 = jnp.where(c1 <= kf, jnp.int32(1), jnp.int32(0))
    lo0 = jnp.maximum(s1lo, one)

    def cond(state):
        lo, hi, done = state
        return jnp.min(done) < 1

    def body(state):
        lo, hi, done = state
        q = (hi - lo) // 4
        m1 = lo + q
        m2 = jnp.maximum(lo + 2 * q, lo + 1)
        m3 = jnp.maximum(lo + 3 * q, m2)
        c1_ = count_ge(m1)
        c2_ = count_ge(m2)
        c3_ = count_ge(m3)
        nd = done < 1
        lo = jnp.where(jnp.logical_and(nd, c1_ >= kf), m1, lo)
        lo = jnp.where(jnp.logical_and(nd, c2_ >= kf), m2, lo)
        lo = jnp.where(jnp.logical_and(nd, c3_ >= kf), m3, lo)
        hi = jnp.where(jnp.logical_and(nd, c3_ < kf), m3, hi)
        hi = jnp.where(jnp.logical_and(nd, c2_ < kf), m2, hi)
        hi = jnp.where(jnp.logical_and(nd, c1_ < kf), m1, hi)
        lo = jnp.where(jnp.logical_and(nd, c1_ == kf), m1, lo)
        lo = jnp.where(jnp.logical_and(nd, c2_ == kf), m2, lo)
        lo = jnp.where(jnp.logical_and(nd, c3_ == kf), m3, lo)
        hit = jnp.logical_or(jnp.logical_or(c1_ == kf, c2_ == kf), c3_ == kf)
        fin = jnp.logical_or(hit, hi - lo <= 1)
        done = jnp.maximum(done, jnp.where(fin, jnp.int32(1), jnp.int32(0)))
        return lo, hi, done

    lo, _, _ = jax.lax.while_loop(cond, body, (lo0, hi0, done0))

    cnt = count_ge(lo)
    ties = jnp.any(cnt > kf)

    t_ref[...] = jax.lax.bitcast_convert_type(lo, jnp.float32)
    j_ref[...] = jnp.full((r, 1), _N, jnp.int32)

    @pl.when(ties)
    def _():
        surplus = cnt > kf
        cgt = count_ge(lo + 1)
        want = jnp.where(surplus, kf - cgt, jnp.float32(_N))
        tie = jnp.logical_and(z == lo, surplus)
        col = jax.lax.broadcasted_iota(jnp.int32, (r, _N), 1)

        def tcount(j):
            m = jnp.logical_and(tie, col < j)
            ind = jnp.where(m, jnp.float32(1.0), jnp.float32(0.0))
            return jnp.sum(ind, axis=1, keepdims=True)

        def tbody(i, st):
            tlo, thi = st
            mid = tlo + (thi - tlo) // 2
            c = tcount(mid)
            small = c <= want
            tlo = jnp.where(small, mid, tlo)
            thi = jnp.where(small, thi, mid)
            return tlo, thi

        jlo0 = jnp.zeros((r, 1), jnp.int32)
        jhi0 = jnp.full((r, 1), _N + 1, jnp.int32)
        jcut, _ = jax.lax.fori_loop(0, 16, tbody, (jlo0, jhi0))
        j_ref[...] = jnp.where(surplus, jcut, jnp.int32(_N))


def _tc_search(x):
    grid = _ROWS // _BLOCK_ROWS
    return pl.pallas_call(
        _search_body,
        grid=(grid,),
        in_specs=[pl.BlockSpec((_BLOCK_ROWS, _N), lambda i: (i, 0))],
        out_specs=[pl.BlockSpec((_BLOCK_ROWS, 1), lambda i: (i, 0)),
                   pl.BlockSpec((_BLOCK_ROWS, 1), lambda i: (i, 0))],
        out_shape=[jax.ShapeDtypeStruct((_ROWS, 1), jnp.float32),
                   jax.ShapeDtypeStruct((_ROWS, 1), jnp.int32)],
    )(x)


_NW = 32
_RPT = _ROWS // _NW
_NV = _N // 16


def _sc_mask(x, thr16, j16):
    mesh = plsc.VectorSubcoreMesh(core_axis_name="c", subcore_axis_name="s")

    @functools.partial(
        pl.kernel,
        mesh=mesh,
        out_type=jax.ShapeDtypeStruct((_ROWS, _N), jnp.float32),
        scratch_types=[
            pltpu.VMEM((_N,), jnp.float32),
            pltpu.VMEM((_N,), jnp.float32),
            pltpu.VMEM((_ROWS * 16,), jnp.float32),
            pltpu.VMEM((_ROWS * 16,), jnp.int32),
        ],
    )
    def k(x_hbm, thr_hbm, j_hbm, out_hbm, row_v, orow_v, thr_v, j_v):
        wid = lax.axis_index("s") * 2 + lax.axis_index("c")
        pltpu.sync_copy(thr_hbm, thr_v)
        pltpu.sync_copy(j_hbm, j_v)
        iota16 = jax.lax.broadcasted_iota(jnp.int32, (16,), 0)
        step16 = jnp.full((16,), 16, jnp.int32)
        for rr in range(_RPT):
            row = wid * _RPT + rr
            pltpu.sync_copy(x_hbm.at[row], row_v)
            tspl = thr_v[pl.ds(row * 16, 16)]
            jspl = j_v[pl.ds(row * 16, 16)]

            def body(i, col):
                v = row_v[pl.ds(i * 16, 16)]
                keep = jnp.logical_or(v > tspl,
                                      jnp.logical_and(v == tspl, col < jspl))
                orow_v[pl.ds(i * 16, 16)] = jnp.where(keep, v,
                                                      jnp.float32(0.0))
                return col + step16

            jax.lax.fori_loop(0, _NV, body, iota16)
            pltpu.sync_copy(orow_v, out_hbm.at[row])

    return k(x, thr16, j16)


def kernel(x):
    thr, jcut = _tc_search(x)
    thr16 = jnp.broadcast_to(thr, (_ROWS, 16)).reshape(_ROWS * 16)
    j16 = jnp.broadcast_to(jcut, (_ROWS, 16)).reshape(_ROWS * 16)
    return _sc_mask(x, thr16, j16)


# speedup vs baseline: 1.0045x; 1.0045x over previous
import functools
import jax
import jax.numpy as jnp
from jax import lax
from jax.experimental import pallas as pl
from jax.experimental.pallas import tpu as pltpu
from jax.experimental.pallas import tpu_sc as plsc

_K = 64
_N = 32768
_ROWS = 128
_CHUNK = 32
_NCH = _ROWS // _CHUNK
_NW = 32
_RPT = _CHUNK // _NW
_NV = _N // 16


def _search_body(x_ref, t_ref, j_ref):
    x = x_ref[...]
    r = x.shape[0]
    xi = jax.lax.bitcast_convert_type(x, jnp.int32)
    z = xi ^ ((xi >> 31) & jnp.int32(0x7FFFFFFF))

    def count_ge(t):
        ind = jnp.where(z >= t, jnp.float32(1.0), jnp.float32(0.0))
        return jnp.sum(ind, axis=1, keepdims=True)

    kf = jnp.float32(_K)
    one = jnp.full((r, 1), 1, jnp.int32)
    c1 = count_ge(one)

    zc = jnp.max(z.reshape(r, _N // 128, 128), axis=1)
    zmax = jnp.max(zc, axis=1, keepdims=True)

    def countc_ge(t):
        ind = jnp.where(zc >= t, jnp.float32(1.0), jnp.float32(0.0))
        return jnp.sum(ind, axis=1, keepdims=True)

    hi0 = jnp.maximum(zmax + 1, one + 1)

    def s1body(i, st):
        lo, hi = st
        q = (hi - lo) // 4
        m1 = lo + q
        m2 = jnp.maximum(lo + 2 * q, lo + 1)
        m3 = jnp.maximum(lo + 3 * q, m2)
        cc1 = countc_ge(m1)
        cc2 = countc_ge(m2)
        cc3 = countc_ge(m3)
        lo = jnp.where(cc1 >= kf, m1, lo)
        lo = jnp.where(cc2 >= kf, m2, lo)
        lo = jnp.where(cc3 >= kf, m3, lo)
        hi = jnp.where(cc3 < kf, m3, hi)
        hi = jnp.where(cc2 < kf, m2, hi)
        hi = jnp.where(cc1 < kf, m1, hi)
        return lo, hi

    s1lo, _ = jax.lax.fori_loop(0, 9, s1body, (one, hi0))

    done0 = jnp.where(c1 <= kf, jnp.int32(1), jnp.int32(0))
    lo0 = jnp.maximum(s1lo, one)

    def cond(state):
        lo, hi, done = state
        return jnp.min(done) < 1

    def body(state):
        lo, hi, done = state
        q = (hi - lo) // 4
        m1 = lo + q
        m2 = jnp.maximum(lo + 2 * q, lo + 1)
        m3 = jnp.maximum(lo + 3 * q, m2)
        c1_ = count_ge(m1)
        c2_ = count_ge(m2)
        c3_ = count_ge(m3)
        nd = done < 1
        lo = jnp.where(jnp.logical_and(nd, c1_ >= kf), m1, lo)
        lo = jnp.where(jnp.logical_and(nd, c2_ >= kf), m2, lo)
        lo = jnp.where(jnp.logical_and(nd, c3_ >= kf), m3, lo)
        hi = jnp.where(jnp.logical_and(nd, c3_ < kf), m3, hi)
        hi = jnp.where(jnp.logical_and(nd, c2_ < kf), m2, hi)
        hi = jnp.where(jnp.logical_and(nd, c1_ < kf), m1, hi)
        lo = jnp.where(jnp.logical_and(nd, c1_ == kf), m1, lo)
        lo = jnp.where(jnp.logical_and(nd, c2_ == kf), m2, lo)
        lo = jnp.where(jnp.logical_and(nd, c3_ == kf), m3, lo)
        hit = jnp.logical_or(jnp.logical_or(c1_ == kf, c2_ == kf), c3_ == kf)
        fin = jnp.logical_or(hit, hi - lo <= 1)
        done = jnp.maximum(done, jnp.where(fin, jnp.int32(1), jnp.int32(0)))
        return lo, hi, done

    lo, _, _ = jax.lax.while_loop(cond, body, (lo0, hi0, done0))

    cnt = count_ge(lo)
    ties = jnp.any(cnt > kf)

    t_ref[...] = jax.lax.bitcast_convert_type(lo, jnp.float32)
    j_ref[...] = jnp.full((r, 1), _N, jnp.int32)

    @pl.when(ties)
    def _():
        surplus = cnt > kf
        cgt = count_ge(lo + 1)
        want = jnp.where(surplus, kf - cgt, jnp.float32(_N))
        tie = jnp.logical_and(z == lo, surplus)
        col = jax.lax.broadcasted_iota(jnp.int32, (r, _N), 1)

        def tcount(j):
            m = jnp.logical_and(tie, col < j)
            ind = jnp.where(m, jnp.float32(1.0), jnp.float32(0.0))
            return jnp.sum(ind, axis=1, keepdims=True)

        def tbody(i, st):
            tlo, thi = st
            mid = tlo + (thi - tlo) // 2
            c = tcount(mid)
            small = c <= want
            tlo = jnp.where(small, mid, tlo)
            thi = jnp.where(small, thi, mid)
            return tlo, thi

        jlo0 = jnp.zeros((r, 1), jnp.int32)
        jhi0 = jnp.full((r, 1), _N + 1, jnp.int32)
        jcut, _ = jax.lax.fori_loop(0, 16, tbody, (jlo0, jhi0))
        j_ref[...] = jnp.where(surplus, jcut, jnp.int32(_N))


def _tc_search_chunk(x, b):
    return pl.pallas_call(
        _search_body,
        grid=(1,),
        in_specs=[pl.BlockSpec((_CHUNK, _N), lambda i, b=b: (b, 0))],
        out_specs=[pl.BlockSpec((_CHUNK, 1), lambda i: (0, 0)),
                   pl.BlockSpec((_CHUNK, 1), lambda i: (0, 0))],
        out_shape=[jax.ShapeDtypeStruct((_CHUNK, 1), jnp.float32),
                   jax.ShapeDtypeStruct((_CHUNK, 1), jnp.int32)],
    )(x)


def _sc_mask_chunk(x, thr16, j16, b):
    mesh = plsc.VectorSubcoreMesh(core_axis_name="c", subcore_axis_name="s")

    @functools.partial(
        pl.kernel,
        mesh=mesh,
        out_type=jax.ShapeDtypeStruct((_CHUNK, _N), jnp.float32),
        scratch_types=[
            pltpu.VMEM((_N,), jnp.float32),
            pltpu.VMEM((_N,), jnp.float32),
            pltpu.VMEM((_CHUNK * 16,), jnp.float32),
            pltpu.VMEM((_CHUNK * 16,), jnp.int32),
        ],
    )
    def k(x_hbm, thr_hbm, j_hbm, out_hbm, row_v, orow_v, thr_v, j_v):
        wid = lax.axis_index("s") * 2 + lax.axis_index("c")
        pltpu.sync_copy(thr_hbm, thr_v)
        pltpu.sync_copy(j_hbm, j_v)
        iota16 = jax.lax.broadcasted_iota(jnp.int32, (16,), 0)
        step16 = jnp.full((16,), 16, jnp.int32)
        row = wid
        src = b * _CHUNK + row
        pltpu.sync_copy(x_hbm.at[src], row_v)
        tspl = thr_v[pl.ds(row * 16, 16)]
        jspl = j_v[pl.ds(row * 16, 16)]

        def body(i, col):
            v = row_v[pl.ds(i * 16, 16)]
            keep = jnp.logical_or(v > tspl,
                                  jnp.logical_and(v == tspl, col < jspl))
            orow_v[pl.ds(i * 16, 16)] = jnp.where(keep, v, jnp.float32(0.0))
            return col + step16

        jax.lax.fori_loop(0, _NV, body, iota16)
        pltpu.sync_copy(orow_v, out_hbm.at[row])

    return k(x, thr16, j16)


def kernel(x):
    outs = []
    for b in range(_NCH):
        thr, jcut = _tc_search_chunk(x, b)
        thr16 = jnp.broadcast_to(thr, (_CHUNK, 16)).reshape(_CHUNK * 16)
        j16 = jnp.broadcast_to(jcut, (_CHUNK, 16)).reshape(_CHUNK * 16)
        outs.append(_sc_mask_chunk(x, thr16, j16, b))
    return jnp.concatenate(outs, axis=0)


# speedup vs baseline: 2.1278x; 2.1183x over previous
import jax
import jax.numpy as jnp
from jax.experimental import pallas as pl

_K = 64
_N = 32768
_ROWS = 128
_BLOCK_ROWS = 32


def _topk_mask_body(x_ref, o_ref):
    x = x_ref[...]
    r = x.shape[0]

    def fkey(t):
        return jax.lax.bitcast_convert_type(t, jnp.float32)

    def count_ge(t):
        ind = jnp.where(x >= fkey(t), jnp.float32(1.0), jnp.float32(0.0))
        return jnp.sum(ind, axis=1, keepdims=True)

    kf = jnp.float32(_K)
    one = jnp.full((r, 1), 1, jnp.int32)
    cpos = jnp.sum(jnp.where(x > jnp.float32(0.0), jnp.float32(1.0),
                             jnp.float32(0.0)), axis=1, keepdims=True)

    xc = jnp.max(x.reshape(r, _N // 128, 128), axis=1)
    xmax = jnp.max(xc, axis=1, keepdims=True)
    zmax = jax.lax.bitcast_convert_type(xmax, jnp.int32)

    def countc_ge(t):
        ind = jnp.where(xc >= fkey(t), jnp.float32(1.0), jnp.float32(0.0))
        return jnp.sum(ind, axis=1, keepdims=True)

    hi0 = jnp.maximum(zmax + 1, one + 1)

    def s1body(i, st):
        lo, hi = st
        q = (hi - lo) // 4
        m1 = lo + q
        m2 = jnp.maximum(lo + 2 * q, lo + 1)
        m3 = jnp.maximum(lo + 3 * q, m2)
        cc1 = countc_ge(m1)
        cc2 = countc_ge(m2)
        cc3 = countc_ge(m3)
        lo = jnp.where(cc1 >= kf, m1, lo)
        lo = jnp.where(cc2 >= kf, m2, lo)
        lo = jnp.where(cc3 >= kf, m3, lo)
        hi = jnp.where(cc3 < kf, m3, hi)
        hi = jnp.where(cc2 < kf, m2, hi)
        hi = jnp.where(cc1 < kf, m1, hi)
        return lo, hi

    s1lo, _ = jax.lax.fori_loop(0, 9, s1body, (one, hi0))

    done0 = jnp.where(cpos <= kf, jnp.int32(1), jnp.int32(0))
    lo0 = jnp.maximum(s1lo, one)

    def cond(state):
        lo, hi, done = state
        return jnp.min(done) < 1

    def body(state):
        lo, hi, done = state
        q = (hi - lo) // 4
        m1 = lo + q
        m2 = jnp.maximum(lo + 2 * q, lo + 1)
        m3 = jnp.maximum(lo + 3 * q, m2)
        c1_ = count_ge(m1)
        c2_ = count_ge(m2)
        c3_ = count_ge(m3)
        nd = done < 1
        lo = jnp.where(jnp.logical_and(nd, c1_ >= kf), m1, lo)
        lo = jnp.where(jnp.logical_and(nd, c2_ >= kf), m2, lo)
        lo = jnp.where(jnp.logical_and(nd, c3_ >= kf), m3, lo)
        hi = jnp.where(jnp.logical_and(nd, c3_ < kf), m3, hi)
        hi = jnp.where(jnp.logical_and(nd, c2_ < kf), m2, hi)
        hi = jnp.where(jnp.logical_and(nd, c1_ < kf), m1, hi)
        lo = jnp.where(jnp.logical_and(nd, c1_ == kf), m1, lo)
        lo = jnp.where(jnp.logical_and(nd, c2_ == kf), m2, lo)
        lo = jnp.where(jnp.logical_and(nd, c3_ == kf), m3, lo)
        hit = jnp.logical_or(jnp.logical_or(c1_ == kf, c2_ == kf), c3_ == kf)
        fin = jnp.logical_or(hit, hi - lo <= 1)
        done = jnp.maximum(done, jnp.where(fin, jnp.int32(1), jnp.int32(0)))
        return lo, hi, done

    lo, _, _ = jax.lax.while_loop(cond, body, (lo0, hi0, done0))

    lof = fkey(lo)
    ind = jnp.where(x >= lof, jnp.float32(1.0), jnp.float32(0.0))
    cnt = jnp.sum(ind, axis=1, keepdims=True)
    o_ref[...] = x * ind
    ties = jnp.any(cnt > kf)

    @pl.when(ties)
    def _():
        surplus = cnt > kf
        cgt = count_ge(lo + 1)
        want = jnp.where(surplus, kf - cgt, jnp.float32(_N))
        tie = jnp.logical_and(x == lof, surplus)
        col = jax.lax.broadcasted_iota(jnp.int32, (r, _N), 1)

        def tcount(j):
            m = jnp.logical_and(tie, col < j)
            indt = jnp.where(m, jnp.float32(1.0), jnp.float32(0.0))
            return jnp.sum(indt, axis=1, keepdims=True)

        def tbody(i, st):
            tlo, thi = st
            mid = tlo + (thi - tlo) // 2
            c = tcount(mid)
            small = c <= want
            tlo = jnp.where(small, mid, tlo)
            thi = jnp.where(small, thi, mid)
            return tlo, thi

        jlo0 = jnp.zeros((r, 1), jnp.int32)
        jhi0 = jnp.full((r, 1), _N + 1, jnp.int32)
        jcut, _ = jax.lax.fori_loop(0, 16, tbody, (jlo0, jhi0))

        ok_tie = jnp.logical_or(x >= fkey(lo + 1),
                                jnp.logical_and(tie, col < jcut))
        keep = jnp.logical_or(jnp.logical_and(surplus, ok_tie),
                              jnp.logical_and(jnp.logical_not(surplus),
                                              x >= lof))
        o_ref[...] = jnp.where(keep, x, jnp.float32(0.0))


def kernel(x):
    grid = _ROWS // _BLOCK_ROWS
    return pl.pallas_call(
        _topk_mask_body,
        grid=(grid,),
        in_specs=[pl.BlockSpec((_BLOCK_ROWS, _N), lambda i: (i, 0))],
        out_specs=pl.BlockSpec((_BLOCK_ROWS, _N), lambda i: (i, 0)),
        out_shape=jax.ShapeDtypeStruct((_ROWS, _N), jnp.float32),
    )(x)


# speedup vs baseline: 2.2302x; 1.0481x over previous
import jax
import jax.numpy as jnp
from jax.experimental import pallas as pl

_K = 64
_N = 32768
_ROWS = 128
_BLOCK_ROWS = 64


def _topk_mask_body(x_ref, o_ref):
    x = x_ref[...]
    r = x.shape[0]

    def fkey(t):
        return jax.lax.bitcast_convert_type(t, jnp.float32)

    def count_ge(t):
        ind = jnp.where(x >= fkey(t), jnp.float32(1.0), jnp.float32(0.0))
        return jnp.sum(ind, axis=1, keepdims=True)

    kf = jnp.float32(_K)
    one = jnp.full((r, 1), 1, jnp.int32)
    cpos = jnp.sum(jnp.where(x > jnp.float32(0.0), jnp.float32(1.0),
                             jnp.float32(0.0)), axis=1, keepdims=True)

    xc = jnp.max(x.reshape(r, _N // 128, 128), axis=1)
    xmax = jnp.max(xc, axis=1, keepdims=True)
    zmax = jax.lax.bitcast_convert_type(xmax, jnp.int32)

    def countc_ge(t):
        ind = jnp.where(xc >= fkey(t), jnp.float32(1.0), jnp.float32(0.0))
        return jnp.sum(ind, axis=1, keepdims=True)

    hi0 = jnp.maximum(zmax + 1, one + 1)

    def s1body(i, st):
        lo, hi = st
        q = (hi - lo) // 4
        m1 = lo + q
        m2 = jnp.maximum(lo + 2 * q, lo + 1)
        m3 = jnp.maximum(lo + 3 * q, m2)
        cc1 = countc_ge(m1)
        cc2 = countc_ge(m2)
        cc3 = countc_ge(m3)
        lo = jnp.where(cc1 >= kf, m1, lo)
        lo = jnp.where(cc2 >= kf, m2, lo)
        lo = jnp.where(cc3 >= kf, m3, lo)
        hi = jnp.where(cc3 < kf, m3, hi)
        hi = jnp.where(cc2 < kf, m2, hi)
        hi = jnp.where(cc1 < kf, m1, hi)
        return lo, hi

    s1lo, _ = jax.lax.fori_loop(0, 9, s1body, (one, hi0))

    done0 = jnp.where(cpos <= kf, jnp.int32(1), jnp.int32(0))
    lo0 = jnp.maximum(s1lo, one)

    def cond(state):
        lo, hi, done = state
        return jnp.min(done) < 1

    def body(state):
        lo, hi, done = state
        q = (hi - lo) // 4
        m1 = lo + q
        m2 = jnp.maximum(lo + 2 * q, lo + 1)
        m3 = jnp.maximum(lo + 3 * q, m2)
        c1_ = count_ge(m1)
        c2_ = count_ge(m2)
        c3_ = count_ge(m3)
        nd = done < 1
        lo = jnp.where(jnp.logical_and(nd, c1_ >= kf), m1, lo)
        lo = jnp.where(jnp.logical_and(nd, c2_ >= kf), m2, lo)
        lo = jnp.where(jnp.logical_and(nd, c3_ >= kf), m3, lo)
        hi = jnp.where(jnp.logical_and(nd, c3_ < kf), m3, hi)
        hi = jnp.where(jnp.logical_and(nd, c2_ < kf), m2, hi)
        hi = jnp.where(jnp.logical_and(nd, c1_ < kf), m1, hi)
        lo = jnp.where(jnp.logical_and(nd, c1_ == kf), m1, lo)
        lo = jnp.where(jnp.logical_and(nd, c2_ == kf), m2, lo)
        lo = jnp.where(jnp.logical_and(nd, c3_ == kf), m3, lo)
        hit = jnp.logical_or(jnp.logical_or(c1_ == kf, c2_ == kf), c3_ == kf)
        fin = jnp.logical_or(hit, hi - lo <= 1)
        done = jnp.maximum(done, jnp.where(fin, jnp.int32(1), jnp.int32(0)))
        return lo, hi, done

    lo, _, _ = jax.lax.while_loop(cond, body, (lo0, hi0, done0))

    lof = fkey(lo)
    ind = jnp.where(x >= lof, jnp.float32(1.0), jnp.float32(0.0))
    cnt = jnp.sum(ind, axis=1, keepdims=True)
    o_ref[...] = x * ind
    ties = jnp.any(cnt > kf)

    @pl.when(ties)
    def _():
        surplus = cnt > kf
        cgt = count_ge(lo + 1)
        want = jnp.where(surplus, kf - cgt, jnp.float32(_N))
        tie = jnp.logical_and(x == lof, surplus)
        col = jax.lax.broadcasted_iota(jnp.int32, (r, _N), 1)

        def tcount(j):
            m = jnp.logical_and(tie, col < j)
            indt = jnp.where(m, jnp.float32(1.0), jnp.float32(0.0))
            return jnp.sum(indt, axis=1, keepdims=True)

        def tbody(i, st):
            tlo, thi = st
            mid = tlo + (thi - tlo) // 2
            c = tcount(mid)
            small = c <= want
            tlo = jnp.where(small, mid, tlo)
            thi = jnp.where(small, thi, mid)
            return tlo, thi

        jlo0 = jnp.zeros((r, 1), jnp.int32)
        jhi0 = jnp.full((r, 1), _N + 1, jnp.int32)
        jcut, _ = jax.lax.fori_loop(0, 16, tbody, (jlo0, jhi0))

        ok_tie = jnp.logical_or(x >= fkey(lo + 1),
                                jnp.logical_and(tie, col < jcut))
        keep = jnp.logical_or(jnp.logical_and(surplus, ok_tie),
                              jnp.logical_and(jnp.logical_not(surplus),
                                              x >= lof))
        o_ref[...] = jnp.where(keep, x, jnp.float32(0.0))


def kernel(x):
    grid = _ROWS // _BLOCK_ROWS
    return pl.pallas_call(
        _topk_mask_body,
        grid=(grid,),
        in_specs=[pl.BlockSpec((_BLOCK_ROWS, _N), lambda i: (i, 0))],
        out_specs=pl.BlockSpec((_BLOCK_ROWS, _N), lambda i: (i, 0)),
        out_shape=jax.ShapeDtypeStruct((_ROWS, _N), jnp.float32),
    )(x)
